# Initial kernel scaffold; baseline (speedup 1.0000x reference)
import functools

import jax
import jax.numpy as jnp
from jax import lax
from jax.experimental import pallas as pl
from jax.experimental.pallas import tpu as pltpu
from jax.experimental.pallas import tpu_sc as plsc

N = 10000
NP = 10240
E = 160000
EP = 163840
CIN = 256
CHID = 512
COUT = 40
COP = 64
NC = 2
NS = 16
CH = 128
STRIPE = NP // NS
MB = 256

_mesh = plsc.VectorSubcoreMesh(
    core_axis_name="c", subcore_axis_name="s", num_cores=NC, num_subcores=NS)


@functools.partial(
    pl.kernel,
    out_type=(jax.ShapeDtypeStruct((NP, 16), jnp.float32),
              jax.ShapeDtypeStruct((NP, 16), jnp.float32)),
    mesh=_mesh,
    scratch_types=[
        pltpu.VMEM((CH,), jnp.int32),
        pltpu.VMEM((CH, 16), jnp.float32),
        pltpu.VMEM((CH, 16), jnp.float32),
        pltpu.VMEM_SHARED((NP, 16), jnp.float32),
    ],
)
def _sc_count(dst_hbm, ones_hbm, zeros_hbm, cnt0_hbm, cnt1_hbm,
              idx_v, ones_v, z_v, slab):
    c = lax.axis_index("c")
    s = lax.axis_index("s")
    pltpu.sync_copy(ones_hbm, ones_v)
    pltpu.sync_copy(zeros_hbm, z_v)
    for k in range(STRIPE // CH):
        pltpu.sync_copy(z_v, slab.at[pl.ds(s * STRIPE + k * CH, CH)])
    plsc.subcore_barrier()

    nchunk = EP // (NC * NS) // CH
    base = (c * NS + s) * (EP // (NC * NS))

    def body(t, carry):
        pltpu.sync_copy(dst_hbm.at[pl.ds(base + t * CH, CH)], idx_v)
        pltpu.sync_copy(ones_v, slab.at[idx_v], add=True)
        return carry

    lax.fori_loop(0, nchunk, body, 0)
    plsc.subcore_barrier()
    st = pl.ds(s * STRIPE, STRIPE)

    @pl.when(c == 0)
    def _():
        pltpu.sync_copy(slab.at[st], cnt0_hbm.at[st])

    @pl.when(c == 1)
    def _():
        pltpu.sync_copy(slab.at[st], cnt1_hbm.at[st])


@functools.partial(
    pl.kernel,
    out_type=(jax.ShapeDtypeStruct((NP, 128), jnp.float32),
              jax.ShapeDtypeStruct((NP, 128), jnp.float32),
              jax.ShapeDtypeStruct((NP, 16), jnp.float32)),
    mesh=_mesh,
    scratch_types=[
        pltpu.VMEM((CH,), jnp.int32),
        pltpu.VMEM((CH,), jnp.int32),
        pltpu.VMEM((CH, 128), jnp.float32),
        pltpu.VMEM((NP,), jnp.float32),
        pltpu.VMEM((CH, 16), jnp.float32),
        pltpu.VMEM_SHARED((NP, 128), jnp.float32),
        pltpu.VMEM_SHARED((NP, 16), jnp.float32),
        pltpu.SemaphoreType.DMA,
    ],
)
def _sc_edge256(xs_lo_hbm, xs_hi_hbm, dinv_hbm, src_hbm, dst_hbm,
                zeros128_hbm, zeros16_hbm,
                px_lo_hbm, px_hi_hbm, pd_hbm,
                src_v, dst_v, rows_v, dinv_v, pd_v, slab, pdslab, sem):
    c = lax.axis_index("c")
    s = lax.axis_index("s")
    pltpu.sync_copy(zeros128_hbm, rows_v)
    for k in range(STRIPE // CH):
        pltpu.sync_copy(rows_v, slab.at[pl.ds(s * STRIPE + k * CH, CH)])

    @pl.when(c == 0)
    def _():
        pltpu.sync_copy(dinv_hbm, dinv_v)
        pltpu.sync_copy(zeros16_hbm, pd_v)
        for k in range(STRIPE // CH):
            pltpu.sync_copy(pd_v, pdslab.at[pl.ds(s * STRIPE + k * CH, CH)])

    plsc.subcore_barrier()

    nchunk = EP // NS // CH
    base = s * (EP // NS)
    lane = lax.iota(jnp.int32, 16)
    zero16 = jnp.zeros((16,), jnp.int32)

    def body(t, carry):
        e0 = base + t * CH
        pltpu.sync_copy(src_hbm.at[pl.ds(e0, CH)], src_v)
        pltpu.sync_copy(dst_hbm.at[pl.ds(e0, CH)], dst_v)

        @pl.when(c == 0)
        def _():
            pltpu.async_copy(xs_lo_hbm.at[src_v], rows_v, sem).wait()
            for g in range(CH // 16):
                sidx = src_v[pl.ds(g * 16, 16)]
                dvals = plsc.load_gather(dinv_v, [sidx])
                plsc.store_scatter(pd_v, [g * 16 + lane, zero16], dvals)
            pltpu.sync_copy(pd_v, pdslab.at[dst_v], add=True)

        @pl.when(c == 1)
        def _():
            pltpu.async_copy(xs_hi_hbm.at[src_v], rows_v, sem).wait()

        pltpu.sync_copy(rows_v, slab.at[dst_v], add=True)
        return carry

    lax.fori_loop(0, nchunk, body, 0)
    plsc.subcore_barrier()
    st = pl.ds(s * STRIPE, STRIPE)

    @pl.when(c == 0)
    def _():
        pltpu.sync_copy(slab.at[st], px_lo_hbm.at[st])
        pltpu.sync_copy(pdslab.at[st], pd_hbm.at[st])

    @pl.when(c == 1)
    def _():
        pltpu.sync_copy(slab.at[st], px_hi_hbm.at[st])


@functools.partial(
    pl.kernel,
    out_type=(jax.ShapeDtypeStruct((NP, COP), jnp.float32),
              jax.ShapeDtypeStruct((NP, COP), jnp.float32)),
    mesh=_mesh,
    scratch_types=[
        pltpu.VMEM((CH,), jnp.int32),
        pltpu.VMEM((CH,), jnp.int32),
        pltpu.VMEM((CH, COP), jnp.float32),
        pltpu.VMEM_SHARED((NP, COP), jnp.float32),
        pltpu.SemaphoreType.DMA,
    ],
)
def _sc_edge64(gs_hbm, src_hbm, dst_hbm, zeros64_hbm,
               pg0_hbm, pg1_hbm,
               src_v, dst_v, rows_v, slab, sem):
    c = lax.axis_index("c")
    s = lax.axis_index("s")
    pltpu.sync_copy(zeros64_hbm, rows_v)
    for k in range(STRIPE // CH):
        pltpu.sync_copy(rows_v, slab.at[pl.ds(s * STRIPE + k * CH, CH)])
    plsc.subcore_barrier()

    nchunk = EP // (NC * NS) // CH
    base = (c * NS + s) * (EP // (NC * NS))

    def body(t, carry):
        e0 = base + t * CH
        pltpu.sync_copy(src_hbm.at[pl.ds(e0, CH)], src_v)
        pltpu.sync_copy(dst_hbm.at[pl.ds(e0, CH)], dst_v)
        pltpu.async_copy(gs_hbm.at[src_v], rows_v, sem).wait()
        pltpu.sync_copy(rows_v, slab.at[dst_v], add=True)
        return carry

    lax.fori_loop(0, nchunk, body, 0)
    plsc.subcore_barrier()
    st = pl.ds(s * STRIPE, STRIPE)

    @pl.when(c == 0)
    def _():
        pltpu.sync_copy(slab.at[st], pg0_hbm.at[st])

    @pl.when(c == 1)
    def _():
        pltpu.sync_copy(slab.at[st], pg1_hbm.at[st])


def _tc_pre_body(cnt0_ref, cnt1_ref, x_ref, dinv_ref, xs_lo_ref, xs_hi_ref):
    cnt = cnt0_ref[:, :1] + cnt1_ref[:, :1] + 1.0
    dinv = lax.rsqrt(cnt)
    dinv_ref[...] = dinv
    xs = x_ref[...] * dinv
    xs_lo_ref[...] = xs[:, :128]
    xs_hi_ref[...] = xs[:, 128:]


def _tc_pre(cnt0, cnt1, xp):
    nb = NP // MB
    return pl.pallas_call(
        _tc_pre_body,
        grid=(nb,),
        in_specs=[
            pl.BlockSpec((MB, 16), lambda i: (i, 0)),
            pl.BlockSpec((MB, 16), lambda i: (i, 0)),
            pl.BlockSpec((MB, CIN), lambda i: (i, 0)),
        ],
        out_specs=[
            pl.BlockSpec((MB, 1), lambda i: (i, 0)),
            pl.BlockSpec((MB, 128), lambda i: (i, 0)),
            pl.BlockSpec((MB, 128), lambda i: (i, 0)),
        ],
        out_shape=[
            jax.ShapeDtypeStruct((NP, 1), jnp.float32),
            jax.ShapeDtypeStruct((NP, 128), jnp.float32),
            jax.ShapeDtypeStruct((NP, 128), jnp.float32),
        ],
    )(cnt0, cnt1, xp)


def _tc_mlp_body(pxlo_ref, pxhi_ref, x_ref, dinv_ref, pd_ref,
                 w1_ref, b1_ref, w2_ref, b2_ref, gs_ref, t2_ref):
    dinv = dinv_ref[...]
    dinv2 = dinv * dinv
    px = jnp.concatenate([pxlo_ref[...], pxhi_ref[...]], axis=1)
    a = px * dinv + x_ref[...] * dinv2
    s = dinv * pd_ref[:, :1] + dinv2
    h = jnp.dot(a, w1_ref[...], preferred_element_type=jnp.float32)
    h = jnp.maximum(h + s * b1_ref[...], 0.0)
    g = jnp.dot(h, w2_ref[...], preferred_element_type=jnp.float32)
    gs_ref[...] = g * dinv
    t2_ref[...] = g * dinv2 + s * b2_ref[...]


def _tc_mlp(pxlo, pxhi, xp, dinv, pd, w1, b1r, w2p, b2r):
    nb = NP // MB
    return pl.pallas_call(
        _tc_mlp_body,
        grid=(nb,),
        in_specs=[
            pl.BlockSpec((MB, 128), lambda i: (i, 0)),
            pl.BlockSpec((MB, 128), lambda i: (i, 0)),
            pl.BlockSpec((MB, CIN), lambda i: (i, 0)),
            pl.BlockSpec((MB, 1), lambda i: (i, 0)),
            pl.BlockSpec((MB, 16), lambda i: (i, 0)),
            pl.BlockSpec((CIN, CHID), lambda i: (0, 0)),
            pl.BlockSpec((1, CHID), lambda i: (0, 0)),
            pl.BlockSpec((CHID, COP), lambda i: (0, 0)),
            pl.BlockSpec((1, COP), lambda i: (0, 0)),
        ],
        out_specs=[
            pl.BlockSpec((MB, COP), lambda i: (i, 0)),
            pl.BlockSpec((MB, COP), lambda i: (i, 0)),
        ],
        out_shape=[
            jax.ShapeDtypeStruct((NP, COP), jnp.float32),
            jax.ShapeDtypeStruct((NP, COP), jnp.float32),
        ],
    )(pxlo, pxhi, xp, dinv, pd, w1, b1r, w2p, b2r)


def _tc_post_body(pg0_ref, pg1_ref, t2_ref, dinv_ref, o_ref):
    o_ref[...] = (pg0_ref[...] + pg1_ref[...]) * dinv_ref[...] + t2_ref[...]


def _tc_post(pg0, pg1, t2, dinv):
    nb = NP // MB
    return pl.pallas_call(
        _tc_post_body,
        grid=(nb,),
        in_specs=[
            pl.BlockSpec((MB, COP), lambda i: (i, 0)),
            pl.BlockSpec((MB, COP), lambda i: (i, 0)),
            pl.BlockSpec((MB, COP), lambda i: (i, 0)),
            pl.BlockSpec((MB, 1), lambda i: (i, 0)),
        ],
        out_specs=pl.BlockSpec((MB, COP), lambda i: (i, 0)),
        out_shape=jax.ShapeDtypeStruct((NP, COP), jnp.float32),
    )(pg0, pg1, t2, dinv)


def kernel(X, edge_index, W1, b1, W2, b2):
    src = edge_index[0]
    dst = edge_index[1]
    pad = EP - E
    srcp = jnp.concatenate([src, jnp.zeros((pad,), jnp.int32)])
    dstp = jnp.concatenate([dst, jnp.full((pad,), N, jnp.int32)])
    xp = jnp.pad(X, ((0, NP - N), (0, 0)))
    w2p = jnp.pad(W2, ((0, 0), (0, COP - COUT)))
    b1r = b1.reshape(1, CHID)
    b2r = jnp.pad(b2, (0, COP - COUT)).reshape(1, COP)

    ones16 = jnp.zeros((CH, 16), jnp.float32).at[:, 0].set(1.0)
    zeros16 = jnp.zeros((CH, 16), jnp.float32)
    zeros64 = jnp.zeros((CH, COP), jnp.float32)
    zeros128 = jnp.zeros((CH, 128), jnp.float32)

    cnt0, cnt1 = _sc_count(dstp, ones16, zeros16)
    dinv, xs_lo, xs_hi = _tc_pre(cnt0, cnt1, xp)
    dinv1d = dinv[:, 0]
    px_lo, px_hi, pd = _sc_edge256(xs_lo, xs_hi, dinv1d, srcp, dstp,
                                   zeros128, zeros16)
    gs, t2 = _tc_mlp(px_lo, px_hi, xp, dinv, pd, W1, b1r, w2p, b2r)
    pg0, pg1 = _sc_edge64(gs, srcp, dstp, zeros64)
    outp = _tc_post(pg0, pg1, t2, dinv)
    return outp[:N, :COUT]


# speedup vs baseline: 6.4493x; 6.4493x over previous
import functools

import jax
import jax.numpy as jnp
from jax import lax
from jax.experimental import pallas as pl
from jax.experimental.pallas import tpu as pltpu
from jax.experimental.pallas import tpu_sc as plsc

N = 10000
NP = 10240
E = 160000
EP = 163840
CIN = 256
CHID = 512
COUT = 40
COP = 64
NC = 2
NS = 16
CH = 128
STRIPE = NP // NS
MB = 256

_mesh = plsc.VectorSubcoreMesh(
    core_axis_name="c", subcore_axis_name="s", num_cores=NC, num_subcores=NS)


@functools.partial(
    pl.kernel,
    out_type=(jax.ShapeDtypeStruct((NP, 16), jnp.float32),
              jax.ShapeDtypeStruct((NP, 16), jnp.float32)),
    mesh=_mesh,
    compiler_params=pltpu.CompilerParams(use_tc_tiling_on_sc=False),
    scratch_types=[
        pltpu.VMEM((CH,), jnp.int32),
        pltpu.VMEM((CH, 16), jnp.float32),
        pltpu.VMEM((CH, 16), jnp.float32),
        pltpu.VMEM_SHARED((NP, 16), jnp.float32),
    ],
)
def _sc_count(dst_hbm, ones_hbm, zeros_hbm, cnt0_hbm, cnt1_hbm,
              idx_v, ones_v, z_v, slab):
    c = lax.axis_index("c")
    s = lax.axis_index("s")
    pltpu.sync_copy(ones_hbm, ones_v)
    pltpu.sync_copy(zeros_hbm, z_v)
    for k in range(STRIPE // CH):
        pltpu.sync_copy(z_v, slab.at[pl.ds(s * STRIPE + k * CH, CH)])
    plsc.subcore_barrier()

    nchunk = EP // (NC * NS) // CH
    base = (c * NS + s) * (EP // (NC * NS))

    def body(t, carry):
        pltpu.sync_copy(dst_hbm.at[pl.ds(base + t * CH, CH)], idx_v)
        pltpu.sync_copy(ones_v, slab.at[idx_v], add=True)
        return carry

    lax.fori_loop(0, nchunk, body, 0)
    plsc.subcore_barrier()
    st = pl.ds(s * STRIPE, STRIPE)

    @pl.when(c == 0)
    def _():
        pltpu.sync_copy(slab.at[st], cnt0_hbm.at[st])

    @pl.when(c == 1)
    def _():
        pltpu.sync_copy(slab.at[st], cnt1_hbm.at[st])


@functools.partial(
    pl.kernel,
    out_type=(jax.ShapeDtypeStruct((NP, 128), jnp.float32),
              jax.ShapeDtypeStruct((NP, 128), jnp.float32),
              jax.ShapeDtypeStruct((NP, 16), jnp.float32)),
    mesh=_mesh,
    compiler_params=pltpu.CompilerParams(use_tc_tiling_on_sc=False),
    scratch_types=[
        pltpu.VMEM((CH,), jnp.int32),
        pltpu.VMEM((CH,), jnp.int32),
        pltpu.VMEM((CH, 128), jnp.float32),
        pltpu.VMEM((CH, 16), jnp.float32),
        pltpu.VMEM_SHARED((NP, 128), jnp.float32),
        pltpu.VMEM_SHARED((NP, 16), jnp.float32),
        pltpu.SemaphoreType.DMA,
    ],
)
def _sc_edge256(xs_lo_hbm, xs_hi_hbm, dinv16_hbm, src_hbm, dst_hbm,
                zeros128_hbm, zeros16_hbm,
                px_lo_hbm, px_hi_hbm, pd_hbm,
                src_v, dst_v, rows_v, pd_v, slab, pdslab, sem):
    c = lax.axis_index("c")
    s = lax.axis_index("s")
    pltpu.sync_copy(zeros128_hbm, rows_v)
    for k in range(STRIPE // CH):
        pltpu.sync_copy(rows_v, slab.at[pl.ds(s * STRIPE + k * CH, CH)])

    @pl.when(c == 0)
    def _():
        pltpu.sync_copy(zeros16_hbm, pd_v)
        for k in range(STRIPE // CH):
            pltpu.sync_copy(pd_v, pdslab.at[pl.ds(s * STRIPE + k * CH, CH)])

    plsc.subcore_barrier()

    nchunk = EP // NS // CH
    base = s * (EP // NS)

    def body(t, carry):
        e0 = base + t * CH
        pltpu.sync_copy(src_hbm.at[pl.ds(e0, CH)], src_v)
        pltpu.sync_copy(dst_hbm.at[pl.ds(e0, CH)], dst_v)

        @pl.when(c == 0)
        def _():
            pltpu.async_copy(xs_lo_hbm.at[src_v], rows_v, sem).wait()
            pltpu.async_copy(dinv16_hbm.at[src_v], pd_v, sem).wait()
            pltpu.sync_copy(pd_v, pdslab.at[dst_v], add=True)

        @pl.when(c == 1)
        def _():
            pltpu.async_copy(xs_hi_hbm.at[src_v], rows_v, sem).wait()

        pltpu.sync_copy(rows_v, slab.at[dst_v], add=True)
        return carry

    lax.fori_loop(0, nchunk, body, 0)
    plsc.subcore_barrier()
    st = pl.ds(s * STRIPE, STRIPE)

    @pl.when(c == 0)
    def _():
        pltpu.sync_copy(slab.at[st], px_lo_hbm.at[st])
        pltpu.sync_copy(pdslab.at[st], pd_hbm.at[st])

    @pl.when(c == 1)
    def _():
        pltpu.sync_copy(slab.at[st], px_hi_hbm.at[st])


@functools.partial(
    pl.kernel,
    out_type=(jax.ShapeDtypeStruct((NP, COP), jnp.float32),
              jax.ShapeDtypeStruct((NP, COP), jnp.float32)),
    mesh=_mesh,
    compiler_params=pltpu.CompilerParams(use_tc_tiling_on_sc=False),
    scratch_types=[
        pltpu.VMEM((CH,), jnp.int32),
        pltpu.VMEM((CH,), jnp.int32),
        pltpu.VMEM((CH, COP), jnp.float32),
        pltpu.VMEM_SHARED((NP, COP), jnp.float32),
        pltpu.SemaphoreType.DMA,
    ],
)
def _sc_edge64(gs_hbm, src_hbm, dst_hbm, zeros64_hbm,
               pg0_hbm, pg1_hbm,
               src_v, dst_v, rows_v, slab, sem):
    c = lax.axis_index("c")
    s = lax.axis_index("s")
    pltpu.sync_copy(zeros64_hbm, rows_v)
    for k in range(STRIPE // CH):
        pltpu.sync_copy(rows_v, slab.at[pl.ds(s * STRIPE + k * CH, CH)])
    plsc.subcore_barrier()

    nchunk = EP // (NC * NS) // CH
    base = (c * NS + s) * (EP // (NC * NS))

    def body(t, carry):
        e0 = base + t * CH
        pltpu.sync_copy(src_hbm.at[pl.ds(e0, CH)], src_v)
        pltpu.sync_copy(dst_hbm.at[pl.ds(e0, CH)], dst_v)
        pltpu.async_copy(gs_hbm.at[src_v], rows_v, sem).wait()
        pltpu.sync_copy(rows_v, slab.at[dst_v], add=True)
        return carry

    lax.fori_loop(0, nchunk, body, 0)
    plsc.subcore_barrier()
    st = pl.ds(s * STRIPE, STRIPE)

    @pl.when(c == 0)
    def _():
        pltpu.sync_copy(slab.at[st], pg0_hbm.at[st])

    @pl.when(c == 1)
    def _():
        pltpu.sync_copy(slab.at[st], pg1_hbm.at[st])


def _tc_pre_body(cnt0_ref, cnt1_ref, x_ref, dinv_ref, dinv16_ref,
                 xs_lo_ref, xs_hi_ref):
    cnt = cnt0_ref[:, :1] + cnt1_ref[:, :1] + 1.0
    dinv = lax.rsqrt(cnt)
    dinv_ref[...] = dinv
    dinv16_ref[...] = jnp.pad(dinv, ((0, 0), (0, 15)))
    xs = x_ref[...] * dinv
    xs_lo_ref[...] = xs[:, :128]
    xs_hi_ref[...] = xs[:, 128:]


def _tc_pre(cnt0, cnt1, xp):
    nb = NP // MB
    return pl.pallas_call(
        _tc_pre_body,
        grid=(nb,),
        in_specs=[
            pl.BlockSpec((MB, 16), lambda i: (i, 0)),
            pl.BlockSpec((MB, 16), lambda i: (i, 0)),
            pl.BlockSpec((MB, CIN), lambda i: (i, 0)),
        ],
        out_specs=[
            pl.BlockSpec((MB, 1), lambda i: (i, 0)),
            pl.BlockSpec((MB, 16), lambda i: (i, 0)),
            pl.BlockSpec((MB, 128), lambda i: (i, 0)),
            pl.BlockSpec((MB, 128), lambda i: (i, 0)),
        ],
        out_shape=[
            jax.ShapeDtypeStruct((NP, 1), jnp.float32),
            jax.ShapeDtypeStruct((NP, 16), jnp.float32),
            jax.ShapeDtypeStruct((NP, 128), jnp.float32),
            jax.ShapeDtypeStruct((NP, 128), jnp.float32),
        ],
    )(cnt0, cnt1, xp)


def _tc_mlp_body(pxlo_ref, pxhi_ref, x_ref, dinv_ref, pd_ref,
                 w1_ref, b1_ref, w2_ref, b2_ref, gs_ref, t2_ref):
    dinv = dinv_ref[...]
    dinv2 = dinv * dinv
    px = jnp.concatenate([pxlo_ref[...], pxhi_ref[...]], axis=1)
    a = px * dinv + x_ref[...] * dinv2
    s = dinv * pd_ref[:, :1] + dinv2
    h = jnp.dot(a, w1_ref[...], preferred_element_type=jnp.float32)
    h = jnp.maximum(h + s * b1_ref[...], 0.0)
    g = jnp.dot(h, w2_ref[...], preferred_element_type=jnp.float32)
    gs_ref[...] = g * dinv
    t2_ref[...] = g * dinv2 + s * b2_ref[...]


def _tc_mlp(pxlo, pxhi, xp, dinv, pd, w1, b1r, w2p, b2r):
    nb = NP // MB
    return pl.pallas_call(
        _tc_mlp_body,
        grid=(nb,),
        in_specs=[
            pl.BlockSpec((MB, 128), lambda i: (i, 0)),
            pl.BlockSpec((MB, 128), lambda i: (i, 0)),
            pl.BlockSpec((MB, CIN), lambda i: (i, 0)),
            pl.BlockSpec((MB, 1), lambda i: (i, 0)),
            pl.BlockSpec((MB, 16), lambda i: (i, 0)),
            pl.BlockSpec((CIN, CHID), lambda i: (0, 0)),
            pl.BlockSpec((1, CHID), lambda i: (0, 0)),
            pl.BlockSpec((CHID, COP), lambda i: (0, 0)),
            pl.BlockSpec((1, COP), lambda i: (0, 0)),
        ],
        out_specs=[
            pl.BlockSpec((MB, COP), lambda i: (i, 0)),
            pl.BlockSpec((MB, COP), lambda i: (i, 0)),
        ],
        out_shape=[
            jax.ShapeDtypeStruct((NP, COP), jnp.float32),
            jax.ShapeDtypeStruct((NP, COP), jnp.float32),
        ],
    )(pxlo, pxhi, xp, dinv, pd, w1, b1r, w2p, b2r)


def _tc_post_body(pg0_ref, pg1_ref, t2_ref, dinv_ref, o_ref):
    o_ref[...] = (pg0_ref[...] + pg1_ref[...]) * dinv_ref[...] + t2_ref[...]


def _tc_post(pg0, pg1, t2, dinv):
    nb = NP // MB
    return pl.pallas_call(
        _tc_post_body,
        grid=(nb,),
        in_specs=[
            pl.BlockSpec((MB, COP), lambda i: (i, 0)),
            pl.BlockSpec((MB, COP), lambda i: (i, 0)),
            pl.BlockSpec((MB, COP), lambda i: (i, 0)),
            pl.BlockSpec((MB, 1), lambda i: (i, 0)),
        ],
        out_specs=pl.BlockSpec((MB, COP), lambda i: (i, 0)),
        out_shape=jax.ShapeDtypeStruct((NP, COP), jnp.float32),
    )(pg0, pg1, t2, dinv)


def kernel(X, edge_index, W1, b1, W2, b2):
    src = edge_index[0]
    dst = edge_index[1]
    pad = EP - E
    srcp = jnp.concatenate([src, jnp.zeros((pad,), jnp.int32)])
    dstp = jnp.concatenate([dst, jnp.full((pad,), N, jnp.int32)])
    xp = jnp.pad(X, ((0, NP - N), (0, 0)))
    w2p = jnp.pad(W2, ((0, 0), (0, COP - COUT)))
    b1r = b1.reshape(1, CHID)
    b2r = jnp.pad(b2, (0, COP - COUT)).reshape(1, COP)

    ones16 = jnp.zeros((CH, 16), jnp.float32).at[:, 0].set(1.0)
    zeros16 = jnp.zeros((CH, 16), jnp.float32)
    zeros64 = jnp.zeros((CH, COP), jnp.float32)
    zeros128 = jnp.zeros((CH, 128), jnp.float32)

    cnt0, cnt1 = _sc_count(dstp, ones16, zeros16)
    dinv, dinv16, xs_lo, xs_hi = _tc_pre(cnt0, cnt1, xp)
    px_lo, px_hi, pd = _sc_edge256(xs_lo, xs_hi, dinv16, srcp, dstp,
                                   zeros128, zeros16)
    gs, t2 = _tc_mlp(px_lo, px_hi, xp, dinv, pd, W1, b1r, w2p, b2r)
    pg0, pg1 = _sc_edge64(gs, srcp, dstp, zeros64)
    outp = _tc_post(pg0, pg1, t2, dinv)
    return outp[:N, :COUT]


# speedup vs baseline: 7.8822x; 1.2222x over previous
import functools

import jax
import jax.numpy as jnp
from jax import lax
from jax.experimental import pallas as pl
from jax.experimental.pallas import tpu as pltpu
from jax.experimental.pallas import tpu_sc as plsc

N = 10000
NP = 10240
E = 160000
EP = 163840
CIN = 256
CHID = 512
COUT = 40
COP = 64
NC = 2
NS = 16
CH = 128
STRIPE = NP // NS
MB = 256

_mesh = plsc.VectorSubcoreMesh(
    core_axis_name="c", subcore_axis_name="s", num_cores=NC, num_subcores=NS)


@functools.partial(
    pl.kernel,
    out_type=(jax.ShapeDtypeStruct((NP, 16), jnp.float32),
              jax.ShapeDtypeStruct((NP, 16), jnp.float32)),
    mesh=_mesh,
    compiler_params=pltpu.CompilerParams(use_tc_tiling_on_sc=False),
    scratch_types=[
        pltpu.VMEM((4, CH), jnp.int32),
        pltpu.VMEM((CH, 16), jnp.float32),
        pltpu.VMEM((CH, 16), jnp.float32),
        pltpu.VMEM_SHARED((NP, 16), jnp.float32),
        [pltpu.SemaphoreType.DMA for _ in range(4)],
    ],
)
def _sc_count(dst2_hbm, ones_hbm, zeros_hbm, cnt0_hbm, cnt1_hbm,
              idx_v, ones_v, z_v, slab, sems):
    c = lax.axis_index("c")
    s = lax.axis_index("s")
    pltpu.sync_copy(ones_hbm, ones_v)
    pltpu.sync_copy(zeros_hbm, z_v)
    for k in range(STRIPE // CH):
        pltpu.sync_copy(z_v, slab.at[pl.ds(s * STRIPE + k * CH, CH)])
    plsc.subcore_barrier()

    nrow = EP // (NC * NS) // CH
    base = (c * NS + s) * nrow

    def body(t, carry):
        pltpu.sync_copy(dst2_hbm.at[pl.ds(base + t * 4, 4)], idx_v)
        for j in range(4):
            pltpu.sync_copy(ones_v, slab.at[idx_v.at[j]], add=True)
        return carry

    lax.fori_loop(0, nrow // 4, body, 0)
    plsc.subcore_barrier()
    st = pl.ds(s * STRIPE, STRIPE)

    @pl.when(c == 0)
    def _():
        pltpu.sync_copy(slab.at[st], cnt0_hbm.at[st])

    @pl.when(c == 1)
    def _():
        pltpu.sync_copy(slab.at[st], cnt1_hbm.at[st])


NB = 4
NB2 = 2


@functools.partial(
    pl.kernel,
    out_type=(jax.ShapeDtypeStruct((NP, 128), jnp.float32),
              jax.ShapeDtypeStruct((NP, 128), jnp.float32),
              jax.ShapeDtypeStruct((NP, 16), jnp.float32)),
    mesh=_mesh,
    compiler_params=pltpu.CompilerParams(use_tc_tiling_on_sc=False),
    scratch_types=[
        pltpu.VMEM((NB2, CH), jnp.int32),
        pltpu.VMEM((NB2, CH), jnp.int32),
        [pltpu.VMEM((CH, 128), jnp.float32) for _ in range(NB2)],
        [pltpu.VMEM((CH, 16), jnp.float32) for _ in range(NB2)],
        pltpu.VMEM_SHARED((NP, 128), jnp.float32),
        pltpu.VMEM_SHARED((NP, 16), jnp.float32),
        [pltpu.SemaphoreType.DMA for _ in range(NB2)],
        [pltpu.SemaphoreType.DMA for _ in range(NB2)],
        pltpu.SemaphoreType.DMA,
    ],
)
def _sc_edge256(xs_lo_hbm, xs_hi_hbm, dinv16_hbm, src2_hbm, dst2_hbm,
                zeros128_hbm, zeros16_hbm,
                px_lo_hbm, px_hi_hbm, pd_hbm,
                src_v, dst_v, rows_v, pd_v, slab, pdslab,
                gsem, ssem, psem):
    c = lax.axis_index("c")
    s = lax.axis_index("s")
    pltpu.sync_copy(zeros128_hbm, rows_v[0])
    for k in range(STRIPE // CH):
        pltpu.sync_copy(rows_v[0], slab.at[pl.ds(s * STRIPE + k * CH, CH)])

    @pl.when(c == 0)
    def _():
        pltpu.sync_copy(zeros16_hbm, pd_v[0])
        for k in range(STRIPE // CH):
            pltpu.sync_copy(pd_v[0], pdslab.at[pl.ds(s * STRIPE + k * CH, CH)])

    plsc.subcore_barrier()

    nrow = EP // NS // CH
    base = s * nrow

    def outer0(t, carry):
        r0 = base + t * NB2
        pltpu.sync_copy(src2_hbm.at[pl.ds(r0, NB2)], src_v)
        pltpu.sync_copy(dst2_hbm.at[pl.ds(r0, NB2)], dst_v)
        gd, pdd = [], []
        for j in range(NB2):
            gd.append(pltpu.async_copy(
                xs_lo_hbm.at[src_v.at[j]], rows_v[j], gsem[j]))
            pdd.append(pltpu.async_copy(
                dinv16_hbm.at[src_v.at[j]], pd_v[j], ssem[j]))
        for j in range(NB2):
            gd[j].wait()
            pltpu.sync_copy(rows_v[j], slab.at[dst_v.at[j]], add=True)
            pdd[j].wait()
            pltpu.sync_copy(pd_v[j], pdslab.at[dst_v.at[j]], add=True)
        return carry

    def outer1(t, carry):
        r0 = base + t * NB2
        pltpu.sync_copy(src2_hbm.at[pl.ds(r0, NB2)], src_v)
        pltpu.sync_copy(dst2_hbm.at[pl.ds(r0, NB2)], dst_v)
        gd = []
        for j in range(NB2):
            gd.append(pltpu.async_copy(
                xs_hi_hbm.at[src_v.at[j]], rows_v[j], gsem[j]))
        for j in range(NB2):
            gd[j].wait()
            pltpu.sync_copy(rows_v[j], slab.at[dst_v.at[j]], add=True)
        return carry

    @pl.when(c == 0)
    def _():
        lax.fori_loop(0, nrow // NB2, outer0, 0)

    @pl.when(c == 1)
    def _():
        lax.fori_loop(0, nrow // NB2, outer1, 0)

    plsc.subcore_barrier()
    st = pl.ds(s * STRIPE, STRIPE)

    @pl.when(c == 0)
    def _():
        pltpu.sync_copy(slab.at[st], px_lo_hbm.at[st])
        pltpu.sync_copy(pdslab.at[st], pd_hbm.at[st])

    @pl.when(c == 1)
    def _():
        pltpu.sync_copy(slab.at[st], px_hi_hbm.at[st])


@functools.partial(
    pl.kernel,
    out_type=(jax.ShapeDtypeStruct((NP, COP), jnp.float32),
              jax.ShapeDtypeStruct((NP, COP), jnp.float32)),
    mesh=_mesh,
    compiler_params=pltpu.CompilerParams(use_tc_tiling_on_sc=False),
    scratch_types=[
        pltpu.VMEM((NB, CH), jnp.int32),
        pltpu.VMEM((NB, CH), jnp.int32),
        [pltpu.VMEM((CH, COP), jnp.float32) for _ in range(NB)],
        pltpu.VMEM_SHARED((NP, COP), jnp.float32),
        [pltpu.SemaphoreType.DMA for _ in range(NB)],
    ],
)
def _sc_edge64(gs_hbm, src2_hbm, dst2_hbm, zeros64_hbm,
               pg0_hbm, pg1_hbm,
               src_v, dst_v, rows_v, slab, gsem):
    c = lax.axis_index("c")
    s = lax.axis_index("s")
    pltpu.sync_copy(zeros64_hbm, rows_v[0])
    for k in range(STRIPE // CH):
        pltpu.sync_copy(rows_v[0], slab.at[pl.ds(s * STRIPE + k * CH, CH)])
    plsc.subcore_barrier()

    nrow = EP // (NC * NS) // CH
    base = (c * NS + s) * nrow

    def body(t, carry):
        r0 = base + t * NB
        pltpu.sync_copy(src2_hbm.at[pl.ds(r0, NB)], src_v)
        pltpu.sync_copy(dst2_hbm.at[pl.ds(r0, NB)], dst_v)
        gd = [pltpu.async_copy(gs_hbm.at[src_v.at[j]], rows_v[j], gsem[j])
              for j in range(NB)]
        for j in range(NB):
            gd[j].wait()
            pltpu.sync_copy(rows_v[j], slab.at[dst_v.at[j]], add=True)
        return carry

    lax.fori_loop(0, nrow // NB, body, 0)
    plsc.subcore_barrier()
    st = pl.ds(s * STRIPE, STRIPE)

    @pl.when(c == 0)
    def _():
        pltpu.sync_copy(slab.at[st], pg0_hbm.at[st])

    @pl.when(c == 1)
    def _():
        pltpu.sync_copy(slab.at[st], pg1_hbm.at[st])


def _tc_pre_body(cnt0_ref, cnt1_ref, x_ref, dinv_ref, dinv16_ref,
                 xs_lo_ref, xs_hi_ref):
    cnt = cnt0_ref[:, :1] + cnt1_ref[:, :1] + 1.0
    dinv = lax.rsqrt(cnt)
    dinv_ref[...] = dinv
    dinv16_ref[...] = jnp.pad(dinv, ((0, 0), (0, 15)))
    xs = x_ref[...] * dinv
    xs_lo_ref[...] = xs[:, :128]
    xs_hi_ref[...] = xs[:, 128:]


def _tc_pre(cnt0, cnt1, xp):
    nb = NP // MB
    return pl.pallas_call(
        _tc_pre_body,
        grid=(nb,),
        in_specs=[
            pl.BlockSpec((MB, 16), lambda i: (i, 0)),
            pl.BlockSpec((MB, 16), lambda i: (i, 0)),
            pl.BlockSpec((MB, CIN), lambda i: (i, 0)),
        ],
        out_specs=[
            pl.BlockSpec((MB, 1), lambda i: (i, 0)),
            pl.BlockSpec((MB, 16), lambda i: (i, 0)),
            pl.BlockSpec((MB, 128), lambda i: (i, 0)),
            pl.BlockSpec((MB, 128), lambda i: (i, 0)),
        ],
        out_shape=[
            jax.ShapeDtypeStruct((NP, 1), jnp.float32),
            jax.ShapeDtypeStruct((NP, 16), jnp.float32),
            jax.ShapeDtypeStruct((NP, 128), jnp.float32),
            jax.ShapeDtypeStruct((NP, 128), jnp.float32),
        ],
    )(cnt0, cnt1, xp)


def _tc_mlp_body(pxlo_ref, pxhi_ref, x_ref, dinv_ref, pd_ref,
                 w1_ref, b1_ref, w2_ref, b2_ref, gs_ref, t2_ref):
    dinv = dinv_ref[...]
    dinv2 = dinv * dinv
    px = jnp.concatenate([pxlo_ref[...], pxhi_ref[...]], axis=1)
    a = px * dinv + x_ref[...] * dinv2
    s = dinv * pd_ref[:, :1] + dinv2
    h = jnp.dot(a, w1_ref[...], preferred_element_type=jnp.float32)
    h = jnp.maximum(h + s * b1_ref[...], 0.0)
    g = jnp.dot(h, w2_ref[...], preferred_element_type=jnp.float32)
    gs_ref[...] = g * dinv
    t2_ref[...] = g * dinv2 + s * b2_ref[...]


def _tc_mlp(pxlo, pxhi, xp, dinv, pd, w1, b1r, w2p, b2r):
    nb = NP // MB
    return pl.pallas_call(
        _tc_mlp_body,
        grid=(nb,),
        in_specs=[
            pl.BlockSpec((MB, 128), lambda i: (i, 0)),
            pl.BlockSpec((MB, 128), lambda i: (i, 0)),
            pl.BlockSpec((MB, CIN), lambda i: (i, 0)),
            pl.BlockSpec((MB, 1), lambda i: (i, 0)),
            pl.BlockSpec((MB, 16), lambda i: (i, 0)),
            pl.BlockSpec((CIN, CHID), lambda i: (0, 0)),
            pl.BlockSpec((1, CHID), lambda i: (0, 0)),
            pl.BlockSpec((CHID, COP), lambda i: (0, 0)),
            pl.BlockSpec((1, COP), lambda i: (0, 0)),
        ],
        out_specs=[
            pl.BlockSpec((MB, COP), lambda i: (i, 0)),
            pl.BlockSpec((MB, COP), lambda i: (i, 0)),
        ],
        out_shape=[
            jax.ShapeDtypeStruct((NP, COP), jnp.float32),
            jax.ShapeDtypeStruct((NP, COP), jnp.float32),
        ],
    )(pxlo, pxhi, xp, dinv, pd, w1, b1r, w2p, b2r)


def _tc_post_body(pg0_ref, pg1_ref, t2_ref, dinv_ref, o_ref):
    o_ref[...] = (pg0_ref[...] + pg1_ref[...]) * dinv_ref[...] + t2_ref[...]


def _tc_post(pg0, pg1, t2, dinv):
    nb = NP // MB
    return pl.pallas_call(
        _tc_post_body,
        grid=(nb,),
        in_specs=[
            pl.BlockSpec((MB, COP), lambda i: (i, 0)),
            pl.BlockSpec((MB, COP), lambda i: (i, 0)),
            pl.BlockSpec((MB, COP), lambda i: (i, 0)),
            pl.BlockSpec((MB, 1), lambda i: (i, 0)),
        ],
        out_specs=pl.BlockSpec((MB, COP), lambda i: (i, 0)),
        out_shape=jax.ShapeDtypeStruct((NP, COP), jnp.float32),
    )(pg0, pg1, t2, dinv)


def kernel(X, edge_index, W1, b1, W2, b2):
    src = edge_index[0]
    dst = edge_index[1]
    pad = EP - E
    srcp = jnp.concatenate([src, jnp.zeros((pad,), jnp.int32)])
    dstp = jnp.concatenate([dst, jnp.full((pad,), N, jnp.int32)])
    srcp = srcp.reshape(EP // CH, CH)
    dstp = dstp.reshape(EP // CH, CH)
    xp = jnp.pad(X, ((0, NP - N), (0, 0)))
    w2p = jnp.pad(W2, ((0, 0), (0, COP - COUT)))
    b1r = b1.reshape(1, CHID)
    b2r = jnp.pad(b2, (0, COP - COUT)).reshape(1, COP)

    ones16 = jnp.zeros((CH, 16), jnp.float32).at[:, 0].set(1.0)
    zeros16 = jnp.zeros((CH, 16), jnp.float32)
    zeros64 = jnp.zeros((CH, COP), jnp.float32)
    zeros128 = jnp.zeros((CH, 128), jnp.float32)

    cnt0, cnt1 = _sc_count(dstp, ones16, zeros16)
    dinv, dinv16, xs_lo, xs_hi = _tc_pre(cnt0, cnt1, xp)
    px_lo, px_hi, pd = _sc_edge256(xs_lo, xs_hi, dinv16, srcp, dstp,
                                   zeros128, zeros16)
    gs, t2 = _tc_mlp(px_lo, px_hi, xp, dinv, pd, W1, b1r, w2p, b2r)
    pg0, pg1 = _sc_edge64(gs, srcp, dstp, zeros64)
    outp = _tc_post(pg0, pg1, t2, dinv)
    return outp[:N, :COUT]


# speedup vs baseline: 7.9197x; 1.0048x over previous
import functools

import jax
import jax.numpy as jnp
from jax import lax
from jax.experimental import pallas as pl
from jax.experimental.pallas import tpu as pltpu
from jax.experimental.pallas import tpu_sc as plsc

N = 10000
NP = 10240
E = 160000
EP = 163840
CIN = 256
CHID = 512
COUT = 40
COP = 64
NC = 2
NS = 16
CH = 128
STRIPE = NP // NS
MB = 256

_mesh = plsc.VectorSubcoreMesh(
    core_axis_name="c", subcore_axis_name="s", num_cores=NC, num_subcores=NS)


@functools.partial(
    pl.kernel,
    out_type=(jax.ShapeDtypeStruct((NP, 16), jnp.float32),
              jax.ShapeDtypeStruct((NP, 16), jnp.float32)),
    mesh=_mesh,
    compiler_params=pltpu.CompilerParams(use_tc_tiling_on_sc=False),
    scratch_types=[
        pltpu.VMEM((4, CH), jnp.int32),
        pltpu.VMEM((CH, 16), jnp.float32),
        pltpu.VMEM((CH, 16), jnp.float32),
        pltpu.VMEM_SHARED((NP, 16), jnp.float32),
        [pltpu.SemaphoreType.DMA for _ in range(4)],
    ],
)
def _sc_count(dst2_hbm, ones_hbm, zeros_hbm, cnt0_hbm, cnt1_hbm,
              idx_v, ones_v, z_v, slab, sems):
    c = lax.axis_index("c")
    s = lax.axis_index("s")
    pltpu.sync_copy(ones_hbm, ones_v)
    pltpu.sync_copy(zeros_hbm, z_v)
    for k in range(STRIPE // CH):
        pltpu.sync_copy(z_v, slab.at[pl.ds(s * STRIPE + k * CH, CH)])
    plsc.subcore_barrier()

    nrow = EP // (NC * NS) // CH
    base = (c * NS + s) * nrow

    def body(t, carry):
        pltpu.sync_copy(dst2_hbm.at[pl.ds(base + t * 4, 4)], idx_v)
        ds = [pltpu.async_copy(ones_v, slab.at[idx_v.at[j]], sems[j],
                               add=True) for j in range(4)]
        for d in ds:
            d.wait()
        return carry

    lax.fori_loop(0, nrow // 4, body, 0)
    plsc.subcore_barrier()
    st = pl.ds(s * STRIPE, STRIPE)

    @pl.when(c == 0)
    def _():
        pltpu.sync_copy(slab.at[st], cnt0_hbm.at[st])

    @pl.when(c == 1)
    def _():
        pltpu.sync_copy(slab.at[st], cnt1_hbm.at[st])


NB = 4
NB2 = 2


@functools.partial(
    pl.kernel,
    out_type=(jax.ShapeDtypeStruct((NP, 128), jnp.float32),
              jax.ShapeDtypeStruct((NP, 128), jnp.float32),
              jax.ShapeDtypeStruct((NP, 16), jnp.float32)),
    mesh=_mesh,
    compiler_params=pltpu.CompilerParams(use_tc_tiling_on_sc=False),
    scratch_types=[
        pltpu.VMEM((NB2, CH), jnp.int32),
        pltpu.VMEM((NB2, CH), jnp.int32),
        [pltpu.VMEM((CH, 128), jnp.float32) for _ in range(NB2)],
        [pltpu.VMEM((CH, 16), jnp.float32) for _ in range(NB2)],
        pltpu.VMEM_SHARED((NP, 128), jnp.float32),
        pltpu.VMEM_SHARED((NP, 16), jnp.float32),
        [pltpu.SemaphoreType.DMA for _ in range(NB2)],
        [pltpu.SemaphoreType.DMA for _ in range(NB2)],
        pltpu.SemaphoreType.DMA,
    ],
)
def _sc_edge256(xs_lo_hbm, xs_hi_hbm, dinv16_hbm, src2_hbm, dst2_hbm,
                zeros128_hbm, zeros16_hbm,
                px_lo_hbm, px_hi_hbm, pd_hbm,
                src_v, dst_v, rows_v, pd_v, slab, pdslab,
                gsem, ssem, psem):
    c = lax.axis_index("c")
    s = lax.axis_index("s")
    pltpu.sync_copy(zeros128_hbm, rows_v[0])
    for k in range(STRIPE // CH):
        pltpu.sync_copy(rows_v[0], slab.at[pl.ds(s * STRIPE + k * CH, CH)])

    @pl.when(c == 0)
    def _():
        pltpu.sync_copy(zeros16_hbm, pd_v[0])
        for k in range(STRIPE // CH):
            pltpu.sync_copy(pd_v[0], pdslab.at[pl.ds(s * STRIPE + k * CH, CH)])

    plsc.subcore_barrier()

    nrow = EP // NS // CH
    base = s * nrow

    def outer0(t, carry):
        r0 = base + t * NB2
        pltpu.sync_copy(src2_hbm.at[pl.ds(r0, NB2)], src_v)
        pltpu.sync_copy(dst2_hbm.at[pl.ds(r0, NB2)], dst_v)
        gd, pdd = [], []
        for j in range(NB2):
            gd.append(pltpu.async_copy(
                xs_lo_hbm.at[src_v.at[j]], rows_v[j], gsem[j]))
            pdd.append(pltpu.async_copy(
                dinv16_hbm.at[src_v.at[j]], pd_v[j], ssem[j]))
        sc = []
        for j in range(NB2):
            gd[j].wait()
            sc.append(pltpu.async_copy(
                rows_v[j], slab.at[dst_v.at[j]], gsem[j], add=True))
            pdd[j].wait()
            sc.append(pltpu.async_copy(
                pd_v[j], pdslab.at[dst_v.at[j]], ssem[j], add=True))
        for d in sc:
            d.wait()
        return carry

    def outer1(t, carry):
        r0 = base + t * NB2
        pltpu.sync_copy(src2_hbm.at[pl.ds(r0, NB2)], src_v)
        pltpu.sync_copy(dst2_hbm.at[pl.ds(r0, NB2)], dst_v)
        gd = []
        for j in range(NB2):
            gd.append(pltpu.async_copy(
                xs_hi_hbm.at[src_v.at[j]], rows_v[j], gsem[j]))
        sc = []
        for j in range(NB2):
            gd[j].wait()
            sc.append(pltpu.async_copy(
                rows_v[j], slab.at[dst_v.at[j]], gsem[j], add=True))
        for d in sc:
            d.wait()
        return carry

    @pl.when(c == 0)
    def _():
        lax.fori_loop(0, nrow // NB2, outer0, 0)

    @pl.when(c == 1)
    def _():
        lax.fori_loop(0, nrow // NB2, outer1, 0)

    plsc.subcore_barrier()
    st = pl.ds(s * STRIPE, STRIPE)

    @pl.when(c == 0)
    def _():
        pltpu.sync_copy(slab.at[st], px_lo_hbm.at[st])
        pltpu.sync_copy(pdslab.at[st], pd_hbm.at[st])

    @pl.when(c == 1)
    def _():
        pltpu.sync_copy(slab.at[st], px_hi_hbm.at[st])


@functools.partial(
    pl.kernel,
    out_type=(jax.ShapeDtypeStruct((NP, COP), jnp.float32),
              jax.ShapeDtypeStruct((NP, COP), jnp.float32)),
    mesh=_mesh,
    compiler_params=pltpu.CompilerParams(use_tc_tiling_on_sc=False),
    scratch_types=[
        pltpu.VMEM((NB, CH), jnp.int32),
        pltpu.VMEM((NB, CH), jnp.int32),
        [pltpu.VMEM((CH, COP), jnp.float32) for _ in range(NB)],
        pltpu.VMEM_SHARED((NP, COP), jnp.float32),
        [pltpu.SemaphoreType.DMA for _ in range(NB)],
    ],
)
def _sc_edge64(gs_hbm, src2_hbm, dst2_hbm, zeros64_hbm,
               pg0_hbm, pg1_hbm,
               src_v, dst_v, rows_v, slab, gsem):
    c = lax.axis_index("c")
    s = lax.axis_index("s")
    pltpu.sync_copy(zeros64_hbm, rows_v[0])
    for k in range(STRIPE // CH):
        pltpu.sync_copy(rows_v[0], slab.at[pl.ds(s * STRIPE + k * CH, CH)])
    plsc.subcore_barrier()

    nrow = EP // (NC * NS) // CH
    base = (c * NS + s) * nrow

    def body(t, carry):
        r0 = base + t * NB
        pltpu.sync_copy(src2_hbm.at[pl.ds(r0, NB)], src_v)
        pltpu.sync_copy(dst2_hbm.at[pl.ds(r0, NB)], dst_v)
        gd = [pltpu.async_copy(gs_hbm.at[src_v.at[j]], rows_v[j], gsem[j])
              for j in range(NB)]
        sc = []
        for j in range(NB):
            gd[j].wait()
            sc.append(pltpu.async_copy(
                rows_v[j], slab.at[dst_v.at[j]], gsem[j], add=True))
        for d in sc:
            d.wait()
        return carry

    lax.fori_loop(0, nrow // NB, body, 0)
    plsc.subcore_barrier()
    st = pl.ds(s * STRIPE, STRIPE)

    @pl.when(c == 0)
    def _():
        pltpu.sync_copy(slab.at[st], pg0_hbm.at[st])

    @pl.when(c == 1)
    def _():
        pltpu.sync_copy(slab.at[st], pg1_hbm.at[st])


def _tc_pre_body(cnt0_ref, cnt1_ref, x_ref, dinv_ref, dinv16_ref,
                 xs_lo_ref, xs_hi_ref):
    cnt = cnt0_ref[:, :1] + cnt1_ref[:, :1] + 1.0
    dinv = lax.rsqrt(cnt)
    dinv_ref[...] = dinv
    dinv16_ref[...] = jnp.pad(dinv, ((0, 0), (0, 15)))
    xs = x_ref[...] * dinv
    xs_lo_ref[...] = xs[:, :128]
    xs_hi_ref[...] = xs[:, 128:]


def _tc_pre(cnt0, cnt1, xp):
    nb = NP // MB
    return pl.pallas_call(
        _tc_pre_body,
        grid=(nb,),
        in_specs=[
            pl.BlockSpec((MB, 16), lambda i: (i, 0)),
            pl.BlockSpec((MB, 16), lambda i: (i, 0)),
            pl.BlockSpec((MB, CIN), lambda i: (i, 0)),
        ],
        out_specs=[
            pl.BlockSpec((MB, 1), lambda i: (i, 0)),
            pl.BlockSpec((MB, 16), lambda i: (i, 0)),
            pl.BlockSpec((MB, 128), lambda i: (i, 0)),
            pl.BlockSpec((MB, 128), lambda i: (i, 0)),
        ],
        out_shape=[
            jax.ShapeDtypeStruct((NP, 1), jnp.float32),
            jax.ShapeDtypeStruct((NP, 16), jnp.float32),
            jax.ShapeDtypeStruct((NP, 128), jnp.float32),
            jax.ShapeDtypeStruct((NP, 128), jnp.float32),
        ],
    )(cnt0, cnt1, xp)


def _tc_mlp_body(pxlo_ref, pxhi_ref, x_ref, dinv_ref, pd_ref,
                 w1_ref, b1_ref, w2_ref, b2_ref, gs_ref, t2_ref):
    dinv = dinv_ref[...]
    dinv2 = dinv * dinv
    px = jnp.concatenate([pxlo_ref[...], pxhi_ref[...]], axis=1)
    a = px * dinv + x_ref[...] * dinv2
    s = dinv * pd_ref[:, :1] + dinv2
    h = jnp.dot(a, w1_ref[...], preferred_element_type=jnp.float32)
    h = jnp.maximum(h + s * b1_ref[...], 0.0)
    g = jnp.dot(h, w2_ref[...], preferred_element_type=jnp.float32)
    gs_ref[...] = g * dinv
    t2_ref[...] = g * dinv2 + s * b2_ref[...]


def _tc_mlp(pxlo, pxhi, xp, dinv, pd, w1, b1r, w2p, b2r):
    nb = NP // MB
    return pl.pallas_call(
        _tc_mlp_body,
        grid=(nb,),
        in_specs=[
            pl.BlockSpec((MB, 128), lambda i: (i, 0)),
            pl.BlockSpec((MB, 128), lambda i: (i, 0)),
            pl.BlockSpec((MB, CIN), lambda i: (i, 0)),
            pl.BlockSpec((MB, 1), lambda i: (i, 0)),
            pl.BlockSpec((MB, 16), lambda i: (i, 0)),
            pl.BlockSpec((CIN, CHID), lambda i: (0, 0)),
            pl.BlockSpec((1, CHID), lambda i: (0, 0)),
            pl.BlockSpec((CHID, COP), lambda i: (0, 0)),
            pl.BlockSpec((1, COP), lambda i: (0, 0)),
        ],
        out_specs=[
            pl.BlockSpec((MB, COP), lambda i: (i, 0)),
            pl.BlockSpec((MB, COP), lambda i: (i, 0)),
        ],
        out_shape=[
            jax.ShapeDtypeStruct((NP, COP), jnp.float32),
            jax.ShapeDtypeStruct((NP, COP), jnp.float32),
        ],
    )(pxlo, pxhi, xp, dinv, pd, w1, b1r, w2p, b2r)


def _tc_post_body(pg0_ref, pg1_ref, t2_ref, dinv_ref, o_ref):
    o_ref[...] = (pg0_ref[...] + pg1_ref[...]) * dinv_ref[...] + t2_ref[...]


def _tc_post(pg0, pg1, t2, dinv):
    nb = NP // MB
    return pl.pallas_call(
        _tc_post_body,
        grid=(nb,),
        in_specs=[
            pl.BlockSpec((MB, COP), lambda i: (i, 0)),
            pl.BlockSpec((MB, COP), lambda i: (i, 0)),
            pl.BlockSpec((MB, COP), lambda i: (i, 0)),
            pl.BlockSpec((MB, 1), lambda i: (i, 0)),
        ],
        out_specs=pl.BlockSpec((MB, COP), lambda i: (i, 0)),
        out_shape=jax.ShapeDtypeStruct((NP, COP), jnp.float32),
    )(pg0, pg1, t2, dinv)


def kernel(X, edge_index, W1, b1, W2, b2):
    src = edge_index[0]
    dst = edge_index[1]
    pad = EP - E
    srcp = jnp.concatenate([src, jnp.zeros((pad,), jnp.int32)])
    dstp = jnp.concatenate([dst, jnp.full((pad,), N, jnp.int32)])
    srcp = srcp.reshape(EP // CH, CH)
    dstp = dstp.reshape(EP // CH, CH)
    xp = jnp.pad(X, ((0, NP - N), (0, 0)))
    w2p = jnp.pad(W2, ((0, 0), (0, COP - COUT)))
    b1r = b1.reshape(1, CHID)
    b2r = jnp.pad(b2, (0, COP - COUT)).reshape(1, COP)

    ones16 = jnp.zeros((CH, 16), jnp.float32).at[:, 0].set(1.0)
    zeros16 = jnp.zeros((CH, 16), jnp.float32)
    zeros64 = jnp.zeros((CH, COP), jnp.float32)
    zeros128 = jnp.zeros((CH, 128), jnp.float32)

    cnt0, cnt1 = _sc_count(dstp, ones16, zeros16)
    dinv, dinv16, xs_lo, xs_hi = _tc_pre(cnt0, cnt1, xp)
    px_lo, px_hi, pd = _sc_edge256(xs_lo, xs_hi, dinv16, srcp, dstp,
                                   zeros128, zeros16)
    gs, t2 = _tc_mlp(px_lo, px_hi, xp, dinv, pd, W1, b1r, w2p, b2r)
    pg0, pg1 = _sc_edge64(gs, srcp, dstp, zeros64)
    outp = _tc_post(pg0, pg1, t2, dinv)
    return outp[:N, :COUT]
